# baseline (speedup 1.0000x reference)
import jax
import jax.numpy as jnp
from jax import lax
from jax.experimental import pallas as pl
from jax.experimental.pallas import tpu as pltpu
from jax.experimental.pallas import tpu_sc as plsc

BATCH = 16
SEQ = 1024
LATENT = 64
CODES = 1024
B = BATCH * SEQ

NC = 2
NS = 16
NW = NC * NS
BPW = B // NW
CHUNK = 128
NCHUNK = BPW // CHUNK
BPG = 8


def _argmin_body(x_ref, e_ref, idx_ref):
    em = e_ref[...]
    e_sq = jnp.sum(em * em, axis=0, keepdims=True)
    cols = lax.broadcasted_iota(jnp.int32, (1, CODES), 1).astype(jnp.float32)
    for u in range(BPG):
        xt = x_ref[u]
        sim = lax.dot_general(
            xt, em, (((0,), (0,)), ((), ())), preferred_element_type=jnp.float32
        )
        x_sq = jnp.sum(xt * xt, axis=0, keepdims=True).T
        dist = (x_sq + e_sq) - 2.0 * sim
        minval = jnp.min(dist, axis=1, keepdims=True)
        idx = jnp.min(jnp.where(dist == minval, cols, float(CODES)), axis=1)
        idx_ref[pl.ds(u * 8, 8), :] = idx.astype(jnp.int32).reshape(8, 128)


def _sc_gather_body(
    em_hbm, idx_hbm, out_hbm, em_v, idx_v, base_v, tile_v, sem_a, sem_b, sem_o
):
    wid = lax.axis_index("s") * NC + lax.axis_index("c")
    b = wid // 2
    qt0 = (wid % 2) * 4
    sems = [sem_a, sem_b]
    SLAB = 8 * CODES
    cps = {
        j: pltpu.async_copy(
            em_hbm.at[pl.ds(j * SLAB, SLAB)],
            em_v.at[pl.ds(j * SLAB, SLAB)],
            sems[j % 2],
        )
        for j in range(2)
    }
    pltpu.sync_copy(idx_hbm.at[pl.ds(wid * 4, 4)], idx_v)

    @plsc.parallel_loop(0, 32, unroll=4)
    def _(s):
        r = s // 8
        qig = (s % 8) * 16
        idx16 = idx_v[r, pl.ds(qig, 16)]
        base_v[r, pl.ds(qig, 16)] = ((idx16 >> 7) << 10) + (idx16 & 127)

    out_cps = []
    for lt in range(8):
        cps[lt].wait()
        if lt + 2 < 8:
            cps[lt + 2] = pltpu.async_copy(
                em_hbm.at[pl.ds((lt + 2) * SLAB, SLAB)],
                em_v.at[pl.ds((lt + 2) * SLAB, SLAB)],
                sems[lt % 2],
            )

        @plsc.parallel_loop(0, 32, unroll=4)
        def _(s, lt=lt):
            r = s // 8
            qig = (s % 8) * 16
            b16 = base_v[r, pl.ds(qig, 16)]
            gs = [
                plsc.load_gather(em_v, [b16 + (lt * 8192 + li * 128)])
                for li in range(8)
            ]
            for li in range(8):
                tile_v[lt, r, li, pl.ds(qig, 16)] = gs[li]

        out_cps.append(
            pltpu.async_copy(
                tile_v.at[lt], out_hbm.at[b, lt, pl.ds(qt0, 4)], sem_o
            )
        )
    for cp in out_cps:
        cp.wait()


def kernel(x, embeddings):
    xt = jnp.swapaxes(x, 1, 2)
    idx2d = pl.pallas_call(
        _argmin_body,
        grid=(BATCH // BPG,),
        in_specs=[
            pl.BlockSpec((BPG, LATENT, SEQ), lambda i: (i, 0, 0)),
            pl.BlockSpec((LATENT, CODES), lambda i: (0, 0)),
        ],
        out_specs=pl.BlockSpec((8 * BPG, 128), lambda i: (i, 0)),
        out_shape=jax.ShapeDtypeStruct((B // CHUNK, CHUNK), jnp.int32),
    )(xt, embeddings)

    em_flat = embeddings.reshape(8, 8, 8, 128).transpose(0, 2, 1, 3).reshape(-1)

    gather = pl.kernel(
        _sc_gather_body,
        mesh=plsc.VectorSubcoreMesh(core_axis_name="c", subcore_axis_name="s"),
        out_type=jax.ShapeDtypeStruct((BATCH, 8, 8, 8, 128), jnp.float32),
        scratch_types=[
            pltpu.VMEM((LATENT * CODES,), jnp.float32),
            pltpu.VMEM((4, CHUNK), jnp.int32),
            pltpu.VMEM((4, CHUNK), jnp.int32),
            pltpu.VMEM((8, 4, 8, 128), jnp.float32),
            pltpu.SemaphoreType.DMA,
            pltpu.SemaphoreType.DMA,
            pltpu.SemaphoreType.DMA,
        ],
        compiler_params=pltpu.CompilerParams(
            use_tc_tiling_on_sc=False, needs_layout_passes=False
        ),
    )
    out5 = gather(em_flat, idx2d)
    return out5.transpose(0, 2, 4, 1, 3).reshape(BATCH, SEQ, LATENT)


# speedup vs baseline: 1.1224x; 1.1224x over previous
import jax
import jax.numpy as jnp
from jax import lax
from jax.experimental import pallas as pl
from jax.experimental.pallas import tpu as pltpu
from jax.experimental.pallas import tpu_sc as plsc

BATCH = 16
SEQ = 1024
LATENT = 64
CODES = 1024
B = BATCH * SEQ

NC = 2
NS = 16
NW = NC * NS
BPW = B // NW
CHUNK = 128
NCHUNK = BPW // CHUNK
BPG = 8


def _argmin_body(x_ref, e_ref, idx_ref):
    em = e_ref[...]
    e_sq = jnp.sum(em * em, axis=0, keepdims=True)
    cols = lax.broadcasted_iota(jnp.int32, (1, CODES), 1).astype(jnp.float32)
    for u in range(BPG):
        xt = x_ref[u]
        sim = lax.dot_general(
            xt, em, (((0,), (0,)), ((), ())), preferred_element_type=jnp.float32
        )
        x_sq = jnp.sum(xt * xt, axis=0, keepdims=True).T
        dist = (x_sq + e_sq) - 2.0 * sim
        minval = jnp.min(dist, axis=1, keepdims=True)
        idx = jnp.min(jnp.where(dist == minval, cols, float(CODES)), axis=1)
        idx_ref[pl.ds(u * 8, 8), :] = idx.astype(jnp.int32).reshape(8, 128)


def _sc_gather_body(em_hbm, idx_hbm, out_hbm, em_v, idx_v, tile_v, sem_o):
    wid = lax.axis_index("s") * NC + lax.axis_index("c")
    b = wid // 2
    lh = wid % 2
    pltpu.sync_copy(em_hbm.at[pl.ds(lh * 4 * 8192, 4 * 8192)], em_v)
    pltpu.sync_copy(idx_hbm.at[pl.ds(b * 8, 8)], idx_v)

    @plsc.parallel_loop(0, 64, unroll=4)
    def _(s):
        r = s // 8
        qig = (s % 8) * 16
        idx16 = idx_v[r, pl.ds(qig, 16)]
        base16 = ((idx16 >> 7) << 10) + (idx16 & 127)
        for g0 in range(0, 32, 16):
            gs = [
                plsc.load_gather(
                    em_v, [base16 + ((l // 8) * 8192 + (l % 8) * 128)]
                )
                for l in range(g0, g0 + 16)
            ]
            for k, l in enumerate(range(g0, g0 + 16)):
                tile_v[l // 8, r, l % 8, pl.ds(qig, 16)] = gs[k]

    out_cps = [
        pltpu.async_copy(
            tile_v.at[lt], out_hbm.at[b, lh * 4 + lt], sem_o
        )
        for lt in range(4)
    ]
    for cp in out_cps:
        cp.wait()


def kernel(x, embeddings):
    xt = jnp.swapaxes(x, 1, 2)
    idx2d = pl.pallas_call(
        _argmin_body,
        grid=(BATCH // BPG,),
        in_specs=[
            pl.BlockSpec((BPG, LATENT, SEQ), lambda i: (i, 0, 0)),
            pl.BlockSpec((LATENT, CODES), lambda i: (0, 0)),
        ],
        out_specs=pl.BlockSpec((8 * BPG, 128), lambda i: (i, 0)),
        out_shape=jax.ShapeDtypeStruct((B // CHUNK, CHUNK), jnp.int32),
    )(xt, embeddings)

    em_flat = embeddings.reshape(8, 8, 8, 128).transpose(0, 2, 1, 3).reshape(-1)

    gather = pl.kernel(
        _sc_gather_body,
        mesh=plsc.VectorSubcoreMesh(core_axis_name="c", subcore_axis_name="s"),
        out_type=jax.ShapeDtypeStruct((BATCH, 8, 8, 8, 128), jnp.float32),
        scratch_types=[
            pltpu.VMEM((LATENT * CODES // 2,), jnp.float32),
            pltpu.VMEM((8, CHUNK), jnp.int32),
            pltpu.VMEM((4, 8, 8, 128), jnp.float32),
            pltpu.SemaphoreType.DMA,
        ],
        compiler_params=pltpu.CompilerParams(
            use_tc_tiling_on_sc=False, needs_layout_passes=False
        ),
    )
    out5 = gather(em_flat, idx2d)
    return out5.transpose(0, 2, 4, 1, 3).reshape(BATCH, SEQ, LATENT)


# speedup vs baseline: 1.1913x; 1.0614x over previous
import jax
import jax.numpy as jnp
from jax import lax
from jax.experimental import pallas as pl
from jax.experimental.pallas import tpu as pltpu
from jax.experimental.pallas import tpu_sc as plsc

BATCH = 16
SEQ = 1024
LATENT = 64
CODES = 1024
B = BATCH * SEQ

NC = 2
NS = 16
NW = NC * NS
BPW = B // NW
CHUNK = 128
NCHUNK = BPW // CHUNK
BPG = 8


def _argmin_body(x_ref, e_ref, idx_ref):
    em = e_ref[...]
    e_sq = jnp.sum(em * em, axis=0, keepdims=True)
    cols = lax.broadcasted_iota(jnp.int32, (1, CODES), 1).astype(jnp.float32)
    for u in range(BPG):
        xt = x_ref[u]
        sim = lax.dot_general(
            xt, em, (((0,), (0,)), ((), ())), preferred_element_type=jnp.float32
        )
        x_sq = jnp.sum(xt * xt, axis=0, keepdims=True).T
        dist = (x_sq + e_sq) - 2.0 * sim
        minval = jnp.min(dist, axis=1, keepdims=True)
        idx = jnp.min(jnp.where(dist == minval, cols, float(CODES)), axis=1)
        idx_ref[pl.ds(u * 8, 8), :] = idx.astype(jnp.int32).reshape(8, 128)


def _sc_gather_body(em_hbm, idx_hbm, out_hbm, em_v, idx_v, tile_v, sem_o):
    wid = lax.axis_index("s") * NC + lax.axis_index("c")
    b2 = wid // 4
    lq = wid % 4
    pltpu.sync_copy(em_hbm.at[pl.ds(lq * 2 * 8192, 2 * 8192)], em_v)
    pltpu.sync_copy(idx_hbm.at[pl.ds(b2 * 16, 16)], idx_v)

    @plsc.parallel_loop(0, 128, unroll=4)
    def _(s):
        r = s // 8
        qig = (s % 8) * 16
        idx16 = idx_v[r, pl.ds(qig, 16)]
        base16 = ((idx16 >> 7) << 10) + (idx16 & 127)
        gs = [
            plsc.load_gather(
                em_v, [base16 + ((l // 8) * 8192 + (l % 8) * 128)]
            )
            for l in range(16)
        ]
        for l in range(16):
            tile_v[s // 64, l // 8, (s // 8) % 8, l % 8, pl.ds(qig, 16)] = gs[l]

    out_cps = [
        pltpu.async_copy(
            tile_v.at[bi, lt], out_hbm.at[b2 * 2 + bi, lq * 2 + lt], sem_o
        )
        for bi in range(2)
        for lt in range(2)
    ]
    for cp in out_cps:
        cp.wait()


def kernel(x, embeddings):
    xt = jnp.swapaxes(x, 1, 2)
    idx2d = pl.pallas_call(
        _argmin_body,
        grid=(BATCH // BPG,),
        in_specs=[
            pl.BlockSpec((BPG, LATENT, SEQ), lambda i: (i, 0, 0)),
            pl.BlockSpec((LATENT, CODES), lambda i: (0, 0)),
        ],
        out_specs=pl.BlockSpec((8 * BPG, 128), lambda i: (i, 0)),
        out_shape=jax.ShapeDtypeStruct((B // CHUNK, CHUNK), jnp.int32),
    )(xt, embeddings)

    em_flat = embeddings.reshape(8, 8, 8, 128).transpose(0, 2, 1, 3).reshape(-1)

    gather = pl.kernel(
        _sc_gather_body,
        mesh=plsc.VectorSubcoreMesh(core_axis_name="c", subcore_axis_name="s"),
        out_type=jax.ShapeDtypeStruct((BATCH, 8, 8, 8, 128), jnp.float32),
        scratch_types=[
            pltpu.VMEM((LATENT * CODES // 4,), jnp.float32),
            pltpu.VMEM((16, CHUNK), jnp.int32),
            pltpu.VMEM((2, 2, 8, 8, 128), jnp.float32),
            pltpu.SemaphoreType.DMA,
        ],
        compiler_params=pltpu.CompilerParams(
            use_tc_tiling_on_sc=False, needs_layout_passes=False
        ),
    )
    out5 = gather(em_flat, idx2d)
    return out5.transpose(0, 2, 4, 1, 3).reshape(BATCH, SEQ, LATENT)


# speedup vs baseline: 1.2145x; 1.0195x over previous
import jax
import jax.numpy as jnp
from jax import lax
from jax.experimental import pallas as pl
from jax.experimental.pallas import tpu as pltpu
from jax.experimental.pallas import tpu_sc as plsc

BATCH = 16
SEQ = 1024
LATENT = 64
CODES = 1024
B = BATCH * SEQ

NC = 2
NS = 16
NW = NC * NS
BPW = B // NW
CHUNK = 128
NCHUNK = BPW // CHUNK
BPG = 8


def _argmin_body(x_ref, e_ref, idx_ref):
    em = e_ref[...]
    e_sq = jnp.sum(em * em, axis=0, keepdims=True)
    cols = lax.broadcasted_iota(jnp.int32, (1, CODES), 1).astype(jnp.float32)
    for u in range(BPG):
        xt = x_ref[u]
        sim = lax.dot_general(
            xt, em, (((0,), (0,)), ((), ())), preferred_element_type=jnp.float32
        )
        x_sq = jnp.sum(xt * xt, axis=0, keepdims=True).T
        dist = (x_sq + e_sq) - 2.0 * sim
        minval = jnp.min(dist, axis=1, keepdims=True)
        idx = jnp.min(jnp.where(dist == minval, cols, float(CODES)), axis=1)
        idx_ref[pl.ds(u * 8, 8), :] = idx.astype(jnp.int32).reshape(8, 128)


def _sc_gather_body(em_hbm, idx_hbm, out_hbm, em_v, idx_v, tile_v, sem_o):
    wid = lax.axis_index("s") * NC + lax.axis_index("c")
    b4 = wid // 8
    le = wid % 8
    pltpu.sync_copy(em_hbm.at[pl.ds(le * 8192, 8192)], em_v)
    pltpu.sync_copy(idx_hbm.at[pl.ds(b4 * 32, 32)], idx_v)

    @plsc.parallel_loop(0, 256, unroll=4)
    def _(s):
        r = s // 8
        qig = (s % 8) * 16
        idx16 = idx_v[r, pl.ds(qig, 16)]
        base16 = ((idx16 >> 7) << 10) + (idx16 & 127)
        gs = [plsc.load_gather(em_v, [base16 + li * 128]) for li in range(8)]
        for li in range(8):
            tile_v[s // 64, (s // 8) % 8, li, pl.ds(qig, 16)] = gs[li]

    out_cps = [
        pltpu.async_copy(
            tile_v.at[bi], out_hbm.at[b4 * 4 + bi, le], sem_o
        )
        for bi in range(4)
    ]
    for cp in out_cps:
        cp.wait()


def kernel(x, embeddings):
    xt = jnp.swapaxes(x, 1, 2)
    idx2d = pl.pallas_call(
        _argmin_body,
        grid=(BATCH // BPG,),
        in_specs=[
            pl.BlockSpec((BPG, LATENT, SEQ), lambda i: (i, 0, 0)),
            pl.BlockSpec((LATENT, CODES), lambda i: (0, 0)),
        ],
        out_specs=pl.BlockSpec((8 * BPG, 128), lambda i: (i, 0)),
        out_shape=jax.ShapeDtypeStruct((B // CHUNK, CHUNK), jnp.int32),
    )(xt, embeddings)

    em_flat = embeddings.reshape(8, 8, 8, 128).transpose(0, 2, 1, 3).reshape(-1)

    gather = pl.kernel(
        _sc_gather_body,
        mesh=plsc.VectorSubcoreMesh(core_axis_name="c", subcore_axis_name="s"),
        out_type=jax.ShapeDtypeStruct((BATCH, 8, 8, 8, 128), jnp.float32),
        scratch_types=[
            pltpu.VMEM((LATENT * CODES // 8,), jnp.float32),
            pltpu.VMEM((32, CHUNK), jnp.int32),
            pltpu.VMEM((4, 8, 8, 128), jnp.float32),
            pltpu.SemaphoreType.DMA,
        ],
        compiler_params=pltpu.CompilerParams(
            use_tc_tiling_on_sc=False, needs_layout_passes=False
        ),
    )
    out5 = gather(em_flat, idx2d)
    return out5.transpose(0, 2, 4, 1, 3).reshape(BATCH, SEQ, LATENT)
